# baseline (speedup 1.0000x reference)
import functools

import jax
import jax.numpy as jnp
from jax import lax
from jax.experimental import pallas as pl
from jax.experimental.pallas import tpu as pltpu
from jax.experimental.pallas import tpu_sc as plsc

B = 16384
D = 64
CH = 128


def _widen_body(x, o):
    o[:, pl.ds(0, D)] = x[...]
    o[:, pl.ds(D, D)] = jnp.zeros_like(x[...])


def _tc_widen(tab, rb):
    n = tab.shape[0]
    return pl.pallas_call(
        _widen_body,
        grid=(n // rb,),
        in_specs=[pl.BlockSpec((rb, D), lambda i: (i, 0))],
        out_specs=pl.BlockSpec((rb, 2 * D), lambda i: (i, 0)),
        out_shape=jax.ShapeDtypeStruct((n, 2 * D), jnp.float32),
    )(tab)


def _sc_gather(Pp, Qp, uidx, pidx):
    info = plsc.get_sparse_core_info()
    NC, NS, L = info.num_cores, info.num_subcores, info.num_lanes
    NW = NC * NS
    bpw = B // NW
    nch = bpw // CH
    mesh = plsc.VectorSubcoreMesh(core_axis_name="c", subcore_axis_name="s")

    u3 = uidx.reshape(NW, nch, CH)
    p3 = pidx.reshape(NW, nch, CH)

    @functools.partial(
        pl.kernel,
        mesh=mesh,
        out_type=[
            jax.ShapeDtypeStruct((B, 2 * D), jnp.float32),
            jax.ShapeDtypeStruct((B, 2 * D), jnp.float32),
        ],
        scratch_types=[
            pltpu.VMEM((nch, CH), jnp.int32),
            pltpu.VMEM((nch, CH), jnp.int32),
            pltpu.VMEM((bpw, 2 * D), jnp.float32),
            pltpu.SemaphoreType.DMA,
        ],
    )
    def k(P_hbm, Q_hbm, u_hbm, pr_hbm, Pout, Qout, uv, pv, buf, sem):
        wid = lax.axis_index("s") * NC + lax.axis_index("c")
        base = wid * bpw
        pltpu.sync_copy(u_hbm.at[wid], uv)
        pltpu.sync_copy(pr_hbm.at[wid], pv)
        for idx, src, out in ((uv, P_hbm, Pout), (pv, Q_hbm, Qout)):
            copies = []
            for c in range(nch):
                copies.append(
                    pltpu.async_copy(
                        src.at[idx.at[c]], buf.at[pl.ds(c * CH, CH)], sem
                    )
                )
            for cp in copies:
                cp.wait()
            pltpu.sync_copy(buf, out.at[pl.ds(base, bpw)])

    return k(Pp, Qp, u3, p3)


def _mlp_body(p, q, w1a, w1b, b1, w2, b2, o):
    lane = lax.broadcasted_iota(jnp.int32, (1, 2 * D), 1)
    keep = lane < D
    pm = jnp.where(keep, p[...], 0.0)
    qm = jnp.where(keep, q[...], 0.0)
    h = jnp.dot(pm, w1a[...], preferred_element_type=jnp.float32)
    h = h + jnp.dot(qm, w1b[...], preferred_element_type=jnp.float32)
    h = jnp.maximum(h + b1[...], 0.0)
    o[...] = jnp.sum(h * w2[...], axis=1, keepdims=True) + b2[...]


def _tc_mlp(P, Q, W1a, W1b, b1r, w2r, b2r):
    TB = 2048
    return pl.pallas_call(
        _mlp_body,
        grid=(B // TB,),
        in_specs=[
            pl.BlockSpec((TB, 2 * D), lambda i: (i, 0)),
            pl.BlockSpec((TB, 2 * D), lambda i: (i, 0)),
            pl.BlockSpec((2 * D, D), lambda i: (0, 0)),
            pl.BlockSpec((2 * D, D), lambda i: (0, 0)),
            pl.BlockSpec((1, D), lambda i: (0, 0)),
            pl.BlockSpec((1, D), lambda i: (0, 0)),
            pl.BlockSpec((1, 1), lambda i: (0, 0)),
        ],
        out_specs=pl.BlockSpec((TB, 1), lambda i: (i, 0)),
        out_shape=jax.ShapeDtypeStruct((B, 1), jnp.float32),
    )(P, Q, W1a, W1b, b1r, w2r, b2r)


def kernel(user, product, P_table, Q_table, W1, b1, W2, b2):
    user = user.astype(jnp.int32)
    product = product.astype(jnp.int32)
    Pp = _tc_widen(P_table, 8000)
    Qp = _tc_widen(Q_table, 4000)
    P, Q = _sc_gather(Pp, Qp, user, product)
    Z = jnp.zeros((D, D), jnp.float32)
    W1a = jnp.concatenate([W1[:D], Z], axis=0)
    W1b = jnp.concatenate([W1[D:], Z], axis=0)
    return _tc_mlp(
        P,
        Q,
        W1a,
        W1b,
        b1.reshape(1, D),
        W2.reshape(1, D),
        b2.reshape(1, 1),
    )


# speedup vs baseline: 1.2511x; 1.2511x over previous
import functools

import jax
import jax.numpy as jnp
from jax import lax
from jax.experimental import pallas as pl
from jax.experimental.pallas import tpu as pltpu
from jax.experimental.pallas import tpu_sc as plsc

B = 16384
D = 64
CH = 128


def _widen_body(x, o):
    xm = x[...].reshape(-1, D)
    o[...] = jnp.concatenate([xm, jnp.zeros_like(xm)], axis=1)


def _tc_widen(tab3, rbg):
    g = tab3.shape[0]
    return pl.pallas_call(
        _widen_body,
        grid=(g // rbg,),
        in_specs=[pl.BlockSpec((rbg, 8, D), lambda i: (i, 0, 0))],
        out_specs=pl.BlockSpec((rbg * 8, 2 * D), lambda i: (i, 0)),
        out_shape=jax.ShapeDtypeStruct((g * 8, 2 * D), jnp.float32),
    )(tab3)


def _sc_gather(Pp, Qp, uidx, pidx):
    info = plsc.get_sparse_core_info()
    NC, NS, L = info.num_cores, info.num_subcores, info.num_lanes
    NW = NC * NS
    bpw = B // NW
    nch = bpw // CH
    mesh = plsc.VectorSubcoreMesh(core_axis_name="c", subcore_axis_name="s")

    u3 = uidx.reshape(NW, nch, CH)
    p3 = pidx.reshape(NW, nch, CH)

    @functools.partial(
        pl.kernel,
        mesh=mesh,
        out_type=[
            jax.ShapeDtypeStruct((B, 2 * D), jnp.float32),
            jax.ShapeDtypeStruct((B, 2 * D), jnp.float32),
        ],
        scratch_types=[
            pltpu.VMEM((nch, CH), jnp.int32),
            pltpu.VMEM((nch, CH), jnp.int32),
            pltpu.VMEM((bpw, 2 * D), jnp.float32),
            pltpu.SemaphoreType.DMA,
        ],
    )
    def k(P_hbm, Q_hbm, u_hbm, pr_hbm, Pout, Qout, uv, pv, buf, sem):
        wid = lax.axis_index("s") * NC + lax.axis_index("c")
        base = wid * bpw
        pltpu.sync_copy(u_hbm.at[wid], uv)
        pltpu.sync_copy(pr_hbm.at[wid], pv)
        for idx, src, out in ((uv, P_hbm, Pout), (pv, Q_hbm, Qout)):
            copies = []
            for c in range(nch):
                copies.append(
                    pltpu.async_copy(
                        src.at[idx.at[c]], buf.at[pl.ds(c * CH, CH)], sem
                    )
                )
            for cp in copies:
                cp.wait()
            pltpu.sync_copy(buf, out.at[pl.ds(base, bpw)])

    return k(Pp, Qp, u3, p3)


def _mlp_body(p, q, w1a, w1b, b1, w2, b2, o):
    lane = lax.broadcasted_iota(jnp.int32, (1, 2 * D), 1)
    keep = lane < D
    pm = jnp.where(keep, p[...], 0.0)
    qm = jnp.where(keep, q[...], 0.0)
    h = jnp.dot(pm, w1a[...], preferred_element_type=jnp.float32)
    h = h + jnp.dot(qm, w1b[...], preferred_element_type=jnp.float32)
    h = jnp.maximum(h + b1[...], 0.0)
    o[...] = jnp.sum(h * w2[...], axis=1, keepdims=True) + b2[...]


def _tc_mlp(P, Q, W1a, W1b, b1r, w2r, b2r):
    TB = 2048
    return pl.pallas_call(
        _mlp_body,
        grid=(B // TB,),
        in_specs=[
            pl.BlockSpec((TB, 2 * D), lambda i: (i, 0)),
            pl.BlockSpec((TB, 2 * D), lambda i: (i, 0)),
            pl.BlockSpec((2 * D, D), lambda i: (0, 0)),
            pl.BlockSpec((2 * D, D), lambda i: (0, 0)),
            pl.BlockSpec((1, D), lambda i: (0, 0)),
            pl.BlockSpec((1, D), lambda i: (0, 0)),
            pl.BlockSpec((1, 1), lambda i: (0, 0)),
        ],
        out_specs=pl.BlockSpec((TB, 1), lambda i: (i, 0)),
        out_shape=jax.ShapeDtypeStruct((B, 1), jnp.float32),
    )(P, Q, W1a, W1b, b1r, w2r, b2r)


def kernel(user, product, P_table, Q_table, W1, b1, W2, b2):
    user = user.astype(jnp.int32)
    product = product.astype(jnp.int32)
    Pp = _tc_widen(P_table.reshape(-1, 8, D), 1000)
    Qp = _tc_widen(Q_table.reshape(-1, 8, D), 1250)
    P, Q = _sc_gather(Pp, Qp, user, product)
    Z = jnp.zeros((D, D), jnp.float32)
    W1a = jnp.concatenate([W1[:D], Z], axis=0)
    W1b = jnp.concatenate([W1[D:], Z], axis=0)
    return _tc_mlp(
        P,
        Q,
        W1a,
        W1b,
        b1.reshape(1, D),
        W2.reshape(1, D),
        b2.reshape(1, 1),
    )
